# baseline (speedup 1.0000x reference)
import functools

import jax
import jax.numpy as jnp
from jax import lax
from jax.experimental import pallas as pl
from jax.experimental.pallas import tpu as pltpu
from jax.experimental.pallas import tpu_sc as plsc

_NT = 21
_RHW = 7
_H = 64
_W = 64
_C = _NT * _RHW * _RHW
_NB = _RHW * _RHW
_NR = 128
_LANES = 16
_NCORES = 2
_NSUB = 16
_NWORK = _NCORES * _NSUB
_CPW = -(-_C // _NWORK)


def _ii_body(fmt_ref, out_ref):
    cb = fmt_ref.shape[2]
    a = jnp.transpose(fmt_ref[...], (2, 0, 1))
    r = lax.broadcasted_iota(jnp.int32, (_W, _W), 0)
    s = lax.broadcasted_iota(jnp.int32, (_W, _W), 1)
    u = (r <= s).astype(jnp.bfloat16)

    def tri_cumsum(m):
        hi = m.astype(jnp.bfloat16)
        lo = (m - hi.astype(jnp.float32)).astype(jnp.bfloat16)
        return (jnp.dot(hi, u, preferred_element_type=jnp.float32)
                + jnp.dot(lo, u, preferred_element_type=jnp.float32))

    b1 = tri_cumsum(a.reshape(cb * _H, _W)).reshape(cb, _H, _W)
    b1t = jnp.swapaxes(b1, 1, 2)
    b2 = tri_cumsum(b1t.reshape(cb * _W, _H))
    out_ref[...] = b2.reshape(cb, _W, _H)


def _idx_body(roist_ref, idx_ref, wgt_ref):
    x1 = jnp.round(roist_ref[0:1, :])
    y1 = jnp.round(roist_ref[1:2, :])
    x2 = jnp.round(roist_ref[2:3, :])
    y2 = jnp.round(roist_ref[3:4, :])
    bin_w = jnp.maximum(x2 - x1 + 1.0, 1.0) / _RHW
    bin_h = jnp.maximum(y2 - y1 + 1.0, 1.0) / _RHW
    p = lax.broadcasted_iota(jnp.int32, (_RHW, _NR), 0).astype(jnp.float32)
    hs = jnp.clip(jnp.floor(p * bin_h) + y1, 0.0, float(_H))
    he = jnp.clip(jnp.ceil((p + 1.0) * bin_h) + y1, 0.0, float(_H))
    ws = jnp.clip(jnp.floor(p * bin_w) + x1, 0.0, float(_W))
    we = jnp.clip(jnp.ceil((p + 1.0) * bin_w) + x1, 0.0, float(_W))
    invc = 1.0 / jnp.maximum((he - hs)[:, None, :] * (we - ws)[None, :, :], 1.0)
    corners = ((he, we, 1.0), (hs, we, -1.0), (he, ws, -1.0), (hs, ws, 1.0))
    idx_parts, wgt_parts = [], []
    for ya, xb, sign in corners:
        yi = ya.astype(jnp.int32)[:, None, :]
        xi = xb.astype(jnp.int32)[None, :, :]
        valid = jnp.logical_and(yi > 0, xi > 0)
        fi = jnp.maximum(xi - 1, 0) * _W + jnp.maximum(yi - 1, 0)
        wk = jnp.where(valid, sign * invc, 0.0)
        idx_parts.append(
            jnp.broadcast_to(fi, (_RHW, _RHW, _NR)).reshape(_NB, _NR))
        wgt_parts.append(wk.reshape(_NB, _NR))
    idx_all = jnp.concatenate(idx_parts, axis=1)
    wgt_all = jnp.concatenate(wgt_parts, axis=1)
    idx_ref[...] = jnp.broadcast_to(
        idx_all[None], (_NT, _NB, 4 * _NR)).reshape(_C, 4 * _NR)
    wgt_ref[...] = jnp.broadcast_to(
        wgt_all[None], (_NT, _NB, 4 * _NR)).reshape(_C, 4 * _NR)


_BCH = 2


def _sc_pool_body(coff, nbatch, ntail,
                  ii_hbm, idx_hbm, wgt_hbm, out_hbm,
                  ii2_v, idx_v, wgt_v, out_v, tail_v, sem0, sem1):
    wid = lax.axis_index("s") * _NCORES + lax.axis_index("c")
    start = wid * (_BCH * nbatch)
    sems = (sem0, sem1)

    nmain = _BCH * nbatch

    def channel_from(img_ref, q, out_row_ref):
        for r0 in range(0, _NR, _LANES):
            acc = jnp.zeros((_LANES,), jnp.float32)
            for k4 in range(4):
                iv = idx_v[q, pl.ds(k4 * _NR + r0, _LANES)]
                ix = lax.shift_right_logical(iv, 6)
                iy = lax.bitwise_and(iv, 63)
                g = plsc.load_gather(img_ref, [ix, iy])
                acc = acc + g * wgt_v[q, pl.ds(k4 * _NR + r0, _LANES)]
            out_row_ref[pl.ds(r0, _LANES)] = acc

    def issue(j, par):
        pltpu.async_copy(
            ii_hbm.at[pl.ds(start + j * _BCH, _BCH)], ii2_v.at[par], sems[par])

    issue(0, 0)
    issue(1, 1)

    pltpu.sync_copy(idx_hbm.at[pl.ds(coff + start, nmain)],
                    idx_v.at[pl.ds(0, nmain)])
    pltpu.sync_copy(wgt_hbm.at[pl.ds(coff + start, nmain)],
                    wgt_v.at[pl.ds(0, nmain)])

    def body(jj, carry):
        for par in range(2):
            j = jj * 2 + par
            c0 = start + j * _BCH
            pltpu.make_async_copy(
                ii_hbm.at[pl.ds(c0, _BCH)], ii2_v.at[par], sems[par]).wait()
            for i in range(_BCH):
                channel_from(ii2_v.at[par, i], j * _BCH + i,
                             out_v.at[j * _BCH + i])

            @pl.when(jj < nbatch // 2 - 1)
            def _():
                issue(j + 2, par)

        return carry

    lax.fori_loop(0, nbatch // 2, body, 0)
    pltpu.sync_copy(out_v, out_hbm.at[pl.ds(start, nmain)])

    if ntail:
        @pl.when(wid < ntail)
        def _():
            c = _NWORK * nmain + wid
            pltpu.sync_copy(ii_hbm.at[c], ii2_v.at[0, 0])
            pltpu.sync_copy(idx_hbm.at[pl.ds(coff + c, 1)],
                            idx_v.at[pl.ds(nmain, 1)])
            pltpu.sync_copy(wgt_hbm.at[pl.ds(coff + c, 1)],
                            wgt_v.at[pl.ds(nmain, 1)])
            channel_from(ii2_v.at[0, 0], nmain, tail_v)
            pltpu.sync_copy(tail_v, out_hbm.at[c])


@functools.cache
def _sc_pool(coff, nch):
    nbatch = nch // (_NWORK * _BCH)
    ntail = nch - _NWORK * _BCH * nbatch
    mesh = plsc.VectorSubcoreMesh(
        core_axis_name="c", subcore_axis_name="s",
        num_cores=_NCORES, num_subcores=_NSUB)
    return pl.kernel(
        functools.partial(_sc_pool_body, coff, nbatch, ntail),
        out_type=jax.ShapeDtypeStruct((nch, _NR), jnp.float32),
        mesh=mesh,
        compiler_params=pltpu.CompilerParams(needs_layout_passes=False),
        scratch_types=[
            pltpu.VMEM((2, _BCH, _W, _H), jnp.float32),
            pltpu.VMEM((_BCH * nbatch + 1, 4 * _NR), jnp.int32),
            pltpu.VMEM((_BCH * nbatch + 1, 4 * _NR), jnp.float32),
            pltpu.VMEM((_BCH * nbatch, _NR), jnp.float32),
            pltpu.VMEM((_NR,), jnp.float32),
            pltpu.SemaphoreType.DMA,
            pltpu.SemaphoreType.DMA,
        ],
    )


def _integral_images(FM, c0, nch):
    fmt = jnp.transpose(FM, (1, 2, 0))
    cb = 128
    nblk = -(-nch // cb)
    blk0 = c0 // cb
    return pl.pallas_call(
        _ii_body,
        grid=(nblk,),
        in_specs=[pl.BlockSpec((_H, _W, cb), lambda i: (0, 0, i + blk0))],
        out_specs=pl.BlockSpec((cb, _W, _H), lambda i: (i, 0, 0)),
        out_shape=jax.ShapeDtypeStruct((nch, _W, _H), jnp.float32),
    )(fmt)


def _corner_tables(rois):
    return pl.pallas_call(
        _idx_body,
        out_shape=(
            jax.ShapeDtypeStruct((_C, 4 * _NR), jnp.int32),
            jax.ShapeDtypeStruct((_C, 4 * _NR), jnp.float32),
        ),
    )(jnp.transpose(rois))


_CHUNKS = (256, 256, 256, 261)


def kernel(FM, rois):
    idxs, wgts = _corner_tables(rois)
    outs = []
    c0 = 0
    for nch in _CHUNKS:
        ii_c = _integral_images(FM, c0, nch)
        outs.append(_sc_pool(c0, nch)(ii_c, idxs, wgts))
        c0 += nch
    out_cr = jnp.concatenate(outs, axis=0)
    return out_cr.reshape(_NT, _RHW, _RHW, _NR).transpose(3, 0, 1, 2)


# speedup vs baseline: 1.0230x; 1.0230x over previous
import functools

import jax
import jax.numpy as jnp
from jax import lax
from jax.experimental import pallas as pl
from jax.experimental.pallas import tpu as pltpu
from jax.experimental.pallas import tpu_sc as plsc

_NT = 21
_RHW = 7
_H = 64
_W = 64
_C = _NT * _RHW * _RHW
_NB = _RHW * _RHW
_NR = 128
_LANES = 16
_NCORES = 2
_NSUB = 16
_NWORK = _NCORES * _NSUB
_CPW = -(-_C // _NWORK)


def _ii_body(fmt_ref, out_ref):
    cb = fmt_ref.shape[2]
    a = jnp.transpose(fmt_ref[...], (2, 0, 1))
    r = lax.broadcasted_iota(jnp.int32, (_W, _W), 0)
    s = lax.broadcasted_iota(jnp.int32, (_W, _W), 1)
    u = (r <= s).astype(jnp.bfloat16)

    def tri_cumsum(m):
        hi = m.astype(jnp.bfloat16)
        lo = (m - hi.astype(jnp.float32)).astype(jnp.bfloat16)
        return (jnp.dot(hi, u, preferred_element_type=jnp.float32)
                + jnp.dot(lo, u, preferred_element_type=jnp.float32))

    b1 = tri_cumsum(a.reshape(cb * _H, _W)).reshape(cb, _H, _W)
    b1t = jnp.swapaxes(b1, 1, 2)
    b2 = tri_cumsum(b1t.reshape(cb * _W, _H))
    out_ref[...] = b2.reshape(cb, _W, _H)


def _idx_body(roist_ref, idx_ref, wgt_ref):
    x1 = jnp.round(roist_ref[0:1, :])
    y1 = jnp.round(roist_ref[1:2, :])
    x2 = jnp.round(roist_ref[2:3, :])
    y2 = jnp.round(roist_ref[3:4, :])
    bin_w = jnp.maximum(x2 - x1 + 1.0, 1.0) / _RHW
    bin_h = jnp.maximum(y2 - y1 + 1.0, 1.0) / _RHW
    p = lax.broadcasted_iota(jnp.int32, (_RHW, _NR), 0).astype(jnp.float32)
    hs = jnp.clip(jnp.floor(p * bin_h) + y1, 0.0, float(_H))
    he = jnp.clip(jnp.ceil((p + 1.0) * bin_h) + y1, 0.0, float(_H))
    ws = jnp.clip(jnp.floor(p * bin_w) + x1, 0.0, float(_W))
    we = jnp.clip(jnp.ceil((p + 1.0) * bin_w) + x1, 0.0, float(_W))
    invc = 1.0 / jnp.maximum((he - hs)[:, None, :] * (we - ws)[None, :, :], 1.0)
    corners = ((he, we, 1.0), (hs, we, -1.0), (he, ws, -1.0), (hs, ws, 1.0))
    idx_parts, wgt_parts = [], []
    for ya, xb, sign in corners:
        yi = ya.astype(jnp.int32)[:, None, :]
        xi = xb.astype(jnp.int32)[None, :, :]
        valid = jnp.logical_and(yi > 0, xi > 0)
        fi = jnp.maximum(xi - 1, 0) * _W + jnp.maximum(yi - 1, 0)
        wk = jnp.where(valid, sign * invc, 0.0)
        idx_parts.append(
            jnp.broadcast_to(fi, (_RHW, _RHW, _NR)).reshape(_NB, _NR))
        wgt_parts.append(wk.reshape(_NB, _NR))
    idx_all = jnp.concatenate(idx_parts, axis=1)
    wgt_all = jnp.concatenate(wgt_parts, axis=1)
    idx_ref[...] = jnp.broadcast_to(
        idx_all[None], (_NT, _NB, 4 * _NR)).reshape(_C, 4 * _NR)
    wgt_ref[...] = jnp.broadcast_to(
        wgt_all[None], (_NT, _NB, 4 * _NR)).reshape(_C, 4 * _NR)


_BCH = 2


def _sc_pool_body(coff, nbatch, ntail,
                  ii_hbm, idx_hbm, wgt_hbm, out_hbm,
                  ii2_v, idx_v, wgt_v, out_v, tail_v, sem0, sem1):
    wid = lax.axis_index("s") * _NCORES + lax.axis_index("c")
    start = wid * (_BCH * nbatch)
    sems = (sem0, sem1)

    nmain = _BCH * nbatch

    def channel_from(img_ref, q, out_row_ref):
        for r0 in range(0, _NR, _LANES):
            acc = jnp.zeros((_LANES,), jnp.float32)
            for k4 in range(4):
                iv = idx_v[q, pl.ds(k4 * _NR + r0, _LANES)]
                ix = lax.shift_right_logical(iv, 6)
                iy = lax.bitwise_and(iv, 63)
                g = plsc.load_gather(img_ref, [ix, iy])
                acc = acc + g * wgt_v[q, pl.ds(k4 * _NR + r0, _LANES)]
            out_row_ref[pl.ds(r0, _LANES)] = acc

    def issue(j, par):
        pltpu.async_copy(
            ii_hbm.at[pl.ds(start + j * _BCH, _BCH)], ii2_v.at[par], sems[par])

    issue(0, 0)
    issue(1, 1)

    pltpu.sync_copy(idx_hbm.at[pl.ds(coff + start, nmain)],
                    idx_v.at[pl.ds(0, nmain)])
    pltpu.sync_copy(wgt_hbm.at[pl.ds(coff + start, nmain)],
                    wgt_v.at[pl.ds(0, nmain)])

    def body(jj, carry):
        for par in range(2):
            j = jj * 2 + par
            c0 = start + j * _BCH
            pltpu.make_async_copy(
                ii_hbm.at[pl.ds(c0, _BCH)], ii2_v.at[par], sems[par]).wait()
            for i in range(_BCH):
                channel_from(ii2_v.at[par, i], j * _BCH + i,
                             out_v.at[j * _BCH + i])

            @pl.when(jj < nbatch // 2 - 1)
            def _():
                issue(j + 2, par)

        return carry

    lax.fori_loop(0, nbatch // 2, body, 0)
    pltpu.sync_copy(out_v, out_hbm.at[pl.ds(start, nmain)])

    if ntail:
        @pl.when(wid < ntail)
        def _():
            c = _NWORK * nmain + wid
            pltpu.sync_copy(ii_hbm.at[c], ii2_v.at[0, 0])
            pltpu.sync_copy(idx_hbm.at[pl.ds(coff + c, 1)],
                            idx_v.at[pl.ds(nmain, 1)])
            pltpu.sync_copy(wgt_hbm.at[pl.ds(coff + c, 1)],
                            wgt_v.at[pl.ds(nmain, 1)])
            channel_from(ii2_v.at[0, 0], nmain, tail_v)
            pltpu.sync_copy(tail_v, out_hbm.at[c])


@functools.cache
def _sc_pool(coff, nch):
    nbatch = nch // (_NWORK * _BCH)
    ntail = nch - _NWORK * _BCH * nbatch
    mesh = plsc.VectorSubcoreMesh(
        core_axis_name="c", subcore_axis_name="s",
        num_cores=_NCORES, num_subcores=_NSUB)
    return pl.kernel(
        functools.partial(_sc_pool_body, coff, nbatch, ntail),
        out_type=jax.ShapeDtypeStruct((nch, _NR), jnp.float32),
        mesh=mesh,
        compiler_params=pltpu.CompilerParams(needs_layout_passes=False),
        scratch_types=[
            pltpu.VMEM((2, _BCH, _W, _H), jnp.float32),
            pltpu.VMEM((_BCH * nbatch + 1, 4 * _NR), jnp.int32),
            pltpu.VMEM((_BCH * nbatch + 1, 4 * _NR), jnp.float32),
            pltpu.VMEM((_BCH * nbatch, _NR), jnp.float32),
            pltpu.VMEM((_NR,), jnp.float32),
            pltpu.SemaphoreType.DMA,
            pltpu.SemaphoreType.DMA,
        ],
    )


def _integral_images(FM, c0, nch):
    fmt = jnp.transpose(FM, (1, 2, 0))
    cb = 128
    nblk = -(-nch // cb)
    blk0 = c0 // cb
    return pl.pallas_call(
        _ii_body,
        grid=(nblk,),
        in_specs=[pl.BlockSpec((_H, _W, cb), lambda i: (0, 0, i + blk0))],
        out_specs=pl.BlockSpec((cb, _W, _H), lambda i: (i, 0, 0)),
        out_shape=jax.ShapeDtypeStruct((nch, _W, _H), jnp.float32),
    )(fmt)


def _corner_tables(rois):
    return pl.pallas_call(
        _idx_body,
        out_shape=(
            jax.ShapeDtypeStruct((_C, 4 * _NR), jnp.int32),
            jax.ShapeDtypeStruct((_C, 4 * _NR), jnp.float32),
        ),
    )(jnp.transpose(rois))


_CHUNKS = (256, 512, 261)


def kernel(FM, rois):
    idxs, wgts = _corner_tables(rois)
    outs = []
    c0 = 0
    for nch in _CHUNKS:
        ii_c = _integral_images(FM, c0, nch)
        outs.append(_sc_pool(c0, nch)(ii_c, idxs, wgts))
        c0 += nch
    out_cr = jnp.concatenate(outs, axis=0)
    return out_cr.reshape(_NT, _RHW, _RHW, _NR).transpose(3, 0, 1, 2)


# speedup vs baseline: 1.0419x; 1.0185x over previous
import functools

import jax
import jax.numpy as jnp
from jax import lax
from jax.experimental import pallas as pl
from jax.experimental.pallas import tpu as pltpu
from jax.experimental.pallas import tpu_sc as plsc

_NT = 21
_RHW = 7
_H = 64
_W = 64
_C = _NT * _RHW * _RHW
_NB = _RHW * _RHW
_NR = 128
_LANES = 16
_NCORES = 2
_NSUB = 16
_NWORK = _NCORES * _NSUB
_CPW = -(-_C // _NWORK)


def _ii_body(fmt_ref, out_ref):
    cb = fmt_ref.shape[2]
    a = jnp.transpose(fmt_ref[...], (2, 0, 1))
    r = lax.broadcasted_iota(jnp.int32, (_W, _W), 0)
    s = lax.broadcasted_iota(jnp.int32, (_W, _W), 1)
    u = (r <= s).astype(jnp.bfloat16)

    def tri_cumsum(m):
        hi = m.astype(jnp.bfloat16)
        lo = (m - hi.astype(jnp.float32)).astype(jnp.bfloat16)
        return (jnp.dot(hi, u, preferred_element_type=jnp.float32)
                + jnp.dot(lo, u, preferred_element_type=jnp.float32))

    b1 = tri_cumsum(a.reshape(cb * _H, _W)).reshape(cb, _H, _W)
    b1t = jnp.swapaxes(b1, 1, 2)
    b2 = tri_cumsum(b1t.reshape(cb * _W, _H))
    out_ref[...] = b2.reshape(cb, _W, _H)


def _idx_body(roist_ref, idx_ref, wgt_ref):
    x1 = jnp.round(roist_ref[0:1, :])
    y1 = jnp.round(roist_ref[1:2, :])
    x2 = jnp.round(roist_ref[2:3, :])
    y2 = jnp.round(roist_ref[3:4, :])
    bin_w = jnp.maximum(x2 - x1 + 1.0, 1.0) / _RHW
    bin_h = jnp.maximum(y2 - y1 + 1.0, 1.0) / _RHW
    p = lax.broadcasted_iota(jnp.int32, (_RHW, _NR), 0).astype(jnp.float32)
    hs = jnp.clip(jnp.floor(p * bin_h) + y1, 0.0, float(_H))
    he = jnp.clip(jnp.ceil((p + 1.0) * bin_h) + y1, 0.0, float(_H))
    ws = jnp.clip(jnp.floor(p * bin_w) + x1, 0.0, float(_W))
    we = jnp.clip(jnp.ceil((p + 1.0) * bin_w) + x1, 0.0, float(_W))
    invc = 1.0 / jnp.maximum((he - hs)[:, None, :] * (we - ws)[None, :, :], 1.0)
    corners = ((he, we, 1.0), (hs, we, -1.0), (he, ws, -1.0), (hs, ws, 1.0))
    idx_parts, wgt_parts = [], []
    for ya, xb, sign in corners:
        yi = ya.astype(jnp.int32)[:, None, :]
        xi = xb.astype(jnp.int32)[None, :, :]
        valid = jnp.logical_and(yi > 0, xi > 0)
        fi = jnp.maximum(xi - 1, 0) * _W + jnp.maximum(yi - 1, 0)
        wk = jnp.where(valid, sign * invc, 0.0)
        idx_parts.append(
            jnp.broadcast_to(fi, (_RHW, _RHW, _NR)).reshape(_NB, _NR))
        wgt_parts.append(wk.reshape(_NB, _NR))
    idx_all = jnp.concatenate(idx_parts, axis=1)
    wgt_all = jnp.concatenate(wgt_parts, axis=1)
    idx_ref[...] = jnp.broadcast_to(
        idx_all[None], (_NT, _NB, 4 * _NR)).reshape(_C, 4 * _NR)
    wgt_ref[...] = jnp.broadcast_to(
        wgt_all[None], (_NT, _NB, 4 * _NR)).reshape(_C, 4 * _NR)


_BCH = 2


def _sc_pool_body(coff, nbatch, ntail,
                  ii_hbm, idx_hbm, wgt_hbm, out_hbm,
                  ii2_v, idx_v, wgt_v, out_v, tail_v, sem0, sem1):
    wid = lax.axis_index("s") * _NCORES + lax.axis_index("c")
    start = wid * (_BCH * nbatch)
    sems = (sem0, sem1)

    nmain = _BCH * nbatch

    def channel_from(img_ref, q, out_row_ref):
        for r0 in range(0, _NR, _LANES):
            acc = jnp.zeros((_LANES,), jnp.float32)
            for k4 in range(4):
                iv = idx_v[q, pl.ds(k4 * _NR + r0, _LANES)]
                ix = lax.shift_right_logical(iv, 6)
                iy = lax.bitwise_and(iv, 63)
                g = plsc.load_gather(img_ref, [ix, iy])
                acc = acc + g * wgt_v[q, pl.ds(k4 * _NR + r0, _LANES)]
            out_row_ref[pl.ds(r0, _LANES)] = acc

    def issue(j, par):
        pltpu.async_copy(
            ii_hbm.at[pl.ds(start + j * _BCH, _BCH)], ii2_v.at[par], sems[par])

    issue(0, 0)
    issue(1, 1)

    pltpu.sync_copy(idx_hbm.at[pl.ds(coff + start, nmain)],
                    idx_v.at[pl.ds(0, nmain)])
    pltpu.sync_copy(wgt_hbm.at[pl.ds(coff + start, nmain)],
                    wgt_v.at[pl.ds(0, nmain)])

    def body(jj, carry):
        for par in range(2):
            j = jj * 2 + par
            c0 = start + j * _BCH
            pltpu.make_async_copy(
                ii_hbm.at[pl.ds(c0, _BCH)], ii2_v.at[par], sems[par]).wait()
            for i in range(_BCH):
                channel_from(ii2_v.at[par, i], j * _BCH + i,
                             out_v.at[j * _BCH + i])

            @pl.when(jj < nbatch // 2 - 1)
            def _():
                issue(j + 2, par)

        return carry

    lax.fori_loop(0, nbatch // 2, body, 0)
    pltpu.sync_copy(out_v, out_hbm.at[pl.ds(start, nmain)])

    if ntail:
        @pl.when(wid < ntail)
        def _():
            c = _NWORK * nmain + wid
            pltpu.sync_copy(ii_hbm.at[c], ii2_v.at[0, 0])
            pltpu.sync_copy(idx_hbm.at[pl.ds(coff + c, 1)],
                            idx_v.at[pl.ds(nmain, 1)])
            pltpu.sync_copy(wgt_hbm.at[pl.ds(coff + c, 1)],
                            wgt_v.at[pl.ds(nmain, 1)])
            channel_from(ii2_v.at[0, 0], nmain, tail_v)
            pltpu.sync_copy(tail_v, out_hbm.at[c])


@functools.cache
def _sc_pool(coff, nch):
    nbatch = nch // (_NWORK * _BCH)
    ntail = nch - _NWORK * _BCH * nbatch
    mesh = plsc.VectorSubcoreMesh(
        core_axis_name="c", subcore_axis_name="s",
        num_cores=_NCORES, num_subcores=_NSUB)
    return pl.kernel(
        functools.partial(_sc_pool_body, coff, nbatch, ntail),
        out_type=jax.ShapeDtypeStruct((nch, _NR), jnp.float32),
        mesh=mesh,
        compiler_params=pltpu.CompilerParams(needs_layout_passes=False),
        scratch_types=[
            pltpu.VMEM((2, _BCH, _W, _H), jnp.float32),
            pltpu.VMEM((_BCH * nbatch + 1, 4 * _NR), jnp.int32),
            pltpu.VMEM((_BCH * nbatch + 1, 4 * _NR), jnp.float32),
            pltpu.VMEM((_BCH * nbatch, _NR), jnp.float32),
            pltpu.VMEM((_NR,), jnp.float32),
            pltpu.SemaphoreType.DMA,
            pltpu.SemaphoreType.DMA,
        ],
    )


def _integral_images(FM, c0, nch):
    fmt = jnp.transpose(FM, (1, 2, 0))
    cb = 128
    nblk = -(-nch // cb)
    blk0 = c0 // cb
    return pl.pallas_call(
        _ii_body,
        grid=(nblk,),
        in_specs=[pl.BlockSpec((_H, _W, cb), lambda i: (0, 0, i + blk0))],
        out_specs=pl.BlockSpec((cb, _W, _H), lambda i: (i, 0, 0)),
        out_shape=jax.ShapeDtypeStruct((nch, _W, _H), jnp.float32),
    )(fmt)


def _corner_tables(rois):
    return pl.pallas_call(
        _idx_body,
        out_shape=(
            jax.ShapeDtypeStruct((_C, 4 * _NR), jnp.int32),
            jax.ShapeDtypeStruct((_C, 4 * _NR), jnp.float32),
        ),
    )(jnp.transpose(rois))


_CHUNKS = (512, 517)


def kernel(FM, rois):
    idxs, wgts = _corner_tables(rois)
    outs = []
    c0 = 0
    for nch in _CHUNKS:
        ii_c = _integral_images(FM, c0, nch)
        outs.append(_sc_pool(c0, nch)(ii_c, idxs, wgts))
        c0 += nch
    out_cr = jnp.concatenate(outs, axis=0)
    return out_cr.reshape(_NT, _RHW, _RHW, _NR).transpose(3, 0, 1, 2)
